# baseline (speedup 1.0000x reference)
import functools

import jax
import jax.numpy as jnp
from jax import lax
from jax.experimental import pallas as pl
from jax.experimental.pallas import tpu as pltpu
from jax.experimental.pallas import tpu_sc as plsc

_NC, _NS = 2, 16
_NW = _NC * _NS
_L = 16
_NBUF = 4


def _rank_body(x_ref, out_ref, *, n, bb):
    x = x_ref[...]
    xi = x[:, :, None]
    xj = x[:, None, :]
    lt = xj < xi
    le = xj <= xi
    ii = lax.broadcasted_iota(jnp.int32, (bb, n, n), 1)
    jj = lax.broadcasted_iota(jnp.int32, (bb, n, n), 2)
    c = jnp.where(lt | ((jj < ii) & le), 1.0, 0.0)
    out_ref[...] = jnp.sum(c, axis=-1).astype(jnp.int32)


def _rank_call(numbers, bb=8, interpret=False):
    b, n = numbers.shape
    return pl.pallas_call(
        functools.partial(_rank_body, n=n, bb=bb),
        grid=(b // bb,),
        in_specs=[pl.BlockSpec((bb, n), lambda i: (i, 0))],
        out_specs=pl.BlockSpec((bb, n), lambda i: (i, 0)),
        out_shape=jax.ShapeDtypeStruct((b, n), jnp.int32),
        interpret=interpret,
    )(numbers)


def _make_sc_lookup(b, n, d):
    rpw = b // _NW
    nk = (n + _L - 1) // _L
    rem = n - (nk - 1) * _L
    mesh = plsc.VectorSubcoreMesh(
        core_axis_name="c", subcore_axis_name="s",
        num_cores=_NC, num_subcores=_NS,
    )

    @functools.partial(
        pl.kernel,
        out_type=jax.ShapeDtypeStruct((n, d, b), jnp.float32),
        mesh=mesh,
        compiler_params=pltpu.CompilerParams(needs_layout_passes=False),
        scratch_types=[
            pltpu.VMEM((n * d,), jnp.float32),
            pltpu.VMEM((rpw * n + _L,), jnp.int32),
            pltpu.VMEM((n * rpw,), jnp.int32),
            pltpu.VMEM((_NBUF, d, rpw), jnp.float32),
            pltpu.SemaphoreType.DMA,
        ],
    )
    def sc_lookup(rank1, tbl1, out_t, tbl_v, rank_v, sidx_v, blk_v, sem):
        wid = lax.axis_index("s") * _NC + lax.axis_index("c")
        bbase = wid * rpw
        pltpu.sync_copy(tbl1.at[pl.ds(0, n * d)], tbl_v.at[pl.ds(0, n * d)])
        pltpu.sync_copy(
            rank1.at[pl.ds(bbase * n, rpw * n)], rank_v.at[pl.ds(0, rpw * n)]
        )

        iota16 = lax.iota(jnp.int32, _L)

        def inv_body(beta, carry):
            for k in range(nk):
                vals = rank_v[pl.ds(beta * n + k * _L, _L)]
                idx = vals * rpw + beta
                ivec = iota16 + (k * _L)
                if k < nk - 1:
                    plsc.store_scatter(sidx_v, [idx], ivec)
                else:
                    plsc.store_scatter(sidx_v, [idx], ivec, mask=iota16 < rem)
            return carry

        lax.fori_loop(0, rpw, inv_body, 0)

        def drain_one():
            pltpu.make_async_copy(
                out_t.at[0, :, pl.ds(0, rpw)], blk_v.at[0], sem
            ).wait()

        def j_body(j, carry):
            buf = lax.rem(j, _NBUF)
            pl.when(j >= _NBUF)(drain_one)
            base = [
                sidx_v[pl.ds(j * rpw + m * _L, _L)] * d for m in range(rpw // _L)
            ]

            def c_body(c, inner):
                for m in range(rpw // _L):
                    vals = plsc.load_gather(tbl_v, [base[m] + c])
                    blk_v[buf, c, pl.ds(m * _L, _L)] = vals
                return inner

            lax.fori_loop(0, d, c_body, 0)
            pltpu.async_copy(
                blk_v.at[buf], out_t.at[j, :, pl.ds(bbase, rpw)], sem
            )
            return carry

        lax.fori_loop(0, n, j_body, 0)
        for _ in range(_NBUF):
            drain_one()

    return sc_lookup


def kernel(numbers, table):
    b, n = numbers.shape
    _, d = table.shape
    rank = _rank_call(numbers)
    rank1 = rank.reshape(b * n)
    tbl1 = table[:n].reshape(n * d)
    out_t = _make_sc_lookup(b, n, d)(rank1, tbl1)
    return jnp.transpose(out_t, (2, 0, 1))


# speedup vs baseline: 2.4024x; 2.4024x over previous
import functools

import jax
import jax.numpy as jnp
from jax import lax
from jax.experimental import pallas as pl
from jax.experimental.pallas import tpu as pltpu
from jax.experimental.pallas import tpu_sc as plsc

_NC, _NS = 2, 16
_NW = _NC * _NS
_L = 16
_NBUF = 4


def _rank_body(x_ref, out_ref, *, n, bb):
    x = x_ref[...]
    xi = x[:, :, None]
    xj = x[:, None, :]
    lt = xj < xi
    le = xj <= xi
    ii = lax.broadcasted_iota(jnp.int32, (bb, n, n), 1)
    jj = lax.broadcasted_iota(jnp.int32, (bb, n, n), 2)
    c = jnp.where(lt | ((jj < ii) & le), 1.0, 0.0)
    out_ref[...] = jnp.sum(c, axis=-1).astype(jnp.int32)


def _rank_call(numbers, bb=8, interpret=False):
    b, n = numbers.shape
    return pl.pallas_call(
        functools.partial(_rank_body, n=n, bb=bb),
        grid=(b // bb,),
        in_specs=[pl.BlockSpec((bb, n), lambda i: (i, 0))],
        out_specs=pl.BlockSpec((bb, n), lambda i: (i, 0)),
        out_shape=jax.ShapeDtypeStruct((b, n), jnp.int32),
        interpret=interpret,
    )(numbers)


def _make_sc_lookup(b, n, d):
    rpw = b // _NW
    nk = (n + _L - 1) // _L
    rem = n - (nk - 1) * _L
    mesh = plsc.VectorSubcoreMesh(
        core_axis_name="c", subcore_axis_name="s",
        num_cores=_NC, num_subcores=_NS,
    )

    @functools.partial(
        pl.kernel,
        out_type=jax.ShapeDtypeStruct((n, d, b), jnp.float32),
        mesh=mesh,
        compiler_params=pltpu.CompilerParams(needs_layout_passes=False),
        scratch_types=[
            pltpu.VMEM((n * d,), jnp.float32),
            pltpu.VMEM((rpw * n + _L,), jnp.int32),
            pltpu.VMEM((n * rpw,), jnp.int32),
            pltpu.VMEM((_NBUF, d, rpw), jnp.float32),
            pltpu.SemaphoreType.DMA,
        ],
    )
    def sc_lookup(rank1, tbl1, out_t, tbl_v, rank_v, sidx_v, blk_v, sem):
        wid = lax.axis_index("s") * _NC + lax.axis_index("c")
        bbase = wid * rpw
        pltpu.sync_copy(tbl1.at[pl.ds(0, n * d)], tbl_v.at[pl.ds(0, n * d)])
        pltpu.sync_copy(
            rank1.at[pl.ds(bbase * n, rpw * n)], rank_v.at[pl.ds(0, rpw * n)]
        )

        iota16 = lax.iota(jnp.int32, _L)

        @functools.partial(plsc.parallel_loop, 0, rpw, unroll=2)
        def inv_body(beta):
            for k in range(nk):
                vals = rank_v[pl.ds(beta * n + k * _L, _L)]
                idx = vals * rpw + beta
                ivec = iota16 + (k * _L)
                if k < nk - 1:
                    plsc.store_scatter(sidx_v, [idx], ivec)
                else:
                    plsc.store_scatter(sidx_v, [idx], ivec, mask=iota16 < rem)

        def drain_one():
            pltpu.make_async_copy(
                out_t.at[0, :, pl.ds(0, rpw)], blk_v.at[0], sem
            ).wait()

        def j_body(j, carry):
            buf = lax.rem(j, _NBUF)
            pl.when(j >= _NBUF)(drain_one)
            base = [
                sidx_v[pl.ds(j * rpw + m * _L, _L)] * d for m in range(rpw // _L)
            ]

            @functools.partial(plsc.parallel_loop, 0, d, unroll=4)
            def c_body(c):
                for m in range(rpw // _L):
                    vals = plsc.load_gather(tbl_v, [base[m] + c])
                    blk_v[buf, c, pl.ds(m * _L, _L)] = vals
            pltpu.async_copy(
                blk_v.at[buf], out_t.at[j, :, pl.ds(bbase, rpw)], sem
            )
            return carry

        lax.fori_loop(0, n, j_body, 0)
        for _ in range(_NBUF):
            drain_one()

    return sc_lookup


def kernel(numbers, table):
    b, n = numbers.shape
    _, d = table.shape
    rank = _rank_call(numbers)
    rank1 = rank.reshape(b * n)
    tbl1 = table[:n].reshape(n * d)
    out_t = _make_sc_lookup(b, n, d)(rank1, tbl1)
    return jnp.transpose(out_t, (2, 0, 1))
